# baseline (speedup 1.0000x reference)
import functools

import jax
import jax.numpy as jnp
from jax.experimental import pallas as pl
from jax.experimental.pallas import tpu as pltpu


def _support_body(x_ref, w_ref, o_ref):
    o_ref[...] = jnp.dot(x_ref[...], w_ref[...], preferred_element_type=jnp.float32)


def _spmm_body(a_ref, s_ref, o_ref):
    o_ref[...] = jnp.dot(a_ref[...], s_ref[...], preferred_element_type=jnp.float32)


@functools.partial(jax.jit, static_argnames=("block_rows",))
def _graph_conv(input_tensor, adj_mat, weights, block_rows=400):
    n, f_in = input_tensor.shape
    f_out = weights.shape[1]
    support = pl.pallas_call(
        _support_body,
        out_shape=jax.ShapeDtypeStruct((n, f_out), jnp.float32),
    )(input_tensor, weights)
    grid = pl.cdiv(n, block_rows)
    return pl.pallas_call(
        _spmm_body,
        grid=(grid,),
        in_specs=[
            pl.BlockSpec((block_rows, n), lambda i: (i, 0)),
            pl.BlockSpec((n, f_out), lambda i: (0, 0)),
        ],
        out_specs=pl.BlockSpec((block_rows, f_out), lambda i: (i, 0)),
        out_shape=jax.ShapeDtypeStruct((n, f_out), jnp.float32),
        compiler_params=pltpu.CompilerParams(
            dimension_semantics=("arbitrary",),
        ),
    )(adj_mat, support)


def kernel(input_tensor, adj_mat, kernel):
    return _graph_conv(input_tensor, adj_mat, kernel)
